# baseline (speedup 1.0000x reference)
import functools

import jax
import jax.numpy as jnp
from jax import lax
from jax.experimental import pallas as pl
from jax.experimental.pallas import tpu as pltpu
from jax.experimental.pallas import tpu_sc as plsc

NC = 2
NS = 16
NW = 1 * NS
L = 16

DIM = 64
NEG = 20
CHUNK = 128


def _sc_gather_dots(v_pos, u_pos, neg_t, v_emb, u_emb):
    B = v_pos.shape[0]
    bpw = B // NW
    nchunk = bpw // CHUNK
    mesh = plsc.VectorSubcoreMesh(core_axis_name="c", subcore_axis_name="s",
                                  num_cores=1)

    @functools.partial(
        pl.kernel,
        out_type=jax.ShapeDtypeStruct((B * 2 * L,), jnp.float32),
        mesh=mesh,
        compiler_params=pltpu.CompilerParams(use_tc_tiling_on_sc=False),
        scratch_types=[
            pltpu.VMEM((2, CHUNK), jnp.int32),
            pltpu.VMEM((2, CHUNK), jnp.int32),
            pltpu.VMEM((2, NEG, CHUNK), jnp.int32),
            pltpu.VMEM((2, CHUNK, DIM), jnp.float32),
            pltpu.VMEM((2, CHUNK, DIM), jnp.float32),
            pltpu.VMEM((2, CHUNK, DIM), jnp.float32),
            pltpu.VMEM((CHUNK * 2 * L,), jnp.float32),
            pltpu.SemaphoreType.DMA,
            pltpu.SemaphoreType.DMA,
        ],
    )
    def sc_k(vpos_hbm, upos_hbm, negt_hbm, vemb_hbm, uemb_hbm, out_hbm,
             vidx, uidx, nidx, vbuf, ubuf, acc, po, sem0, sem1):
        wid = lax.axis_index("s") * 1 + lax.axis_index("c")
        base_w = wid * bpw
        sems = (sem0, sem1)
        zero = jnp.zeros((L,), jnp.float32)

        def zero_body(c, _):
            for s in range(2):
                for j in range(DIM // L):
                    acc.at[s][c, pl.ds(j * L, L)] = zero
            return 0
        lax.fori_loop(0, CHUNK, zero_body, 0)

        def issue(ci, s):
            base = base_w + ci * CHUNK
            pltpu.sync_copy(vpos_hbm.at[pl.ds(base, CHUNK)], vidx.at[s])
            pltpu.sync_copy(upos_hbm.at[pl.ds(base, CHUNK)], uidx.at[s])
            pltpu.sync_copy(negt_hbm.at[:, pl.ds(base, CHUNK)], nidx.at[s])
            cps = [
                pltpu.async_copy(vemb_hbm.at[vidx.at[s]], vbuf.at[s], sems[s]),
                pltpu.async_copy(uemb_hbm.at[uidx.at[s]], ubuf.at[s], sems[s]),
            ]
            for k in range(NEG):
                cps.append(pltpu.async_copy(
                    uemb_hbm.at[nidx.at[s].at[k]], acc.at[s], sems[s],
                    add=True))
            return cps

        def compute(ci, s):
            base = base_w + ci * CHUNK
            vb, ub, ab = vbuf.at[s], ubuf.at[s], acc.at[s]

            def body(c, _):
                pd = jnp.zeros((L,), jnp.float32)
                nd = jnp.zeros((L,), jnp.float32)
                for j in range(DIM // L):
                    sl = pl.ds(j * L, L)
                    vv = vb[c, sl]
                    pd = pd + vv * ub[c, sl]
                    nd = nd + vv * ab[c, sl]
                    ab[c, sl] = zero
                po.at[pl.ds(c * 2 * L, L)][...] = pd
                po.at[pl.ds(c * 2 * L + L, L)][...] = nd
                return 0
            lax.fori_loop(0, CHUNK, body, 0)
            pltpu.sync_copy(po, out_hbm.at[pl.ds(base * 2 * L,
                                                 CHUNK * 2 * L)])

        pending = issue(0, 0)
        for ci in range(nchunk):
            s = ci % 2
            nxt = None
            if ci + 1 < nchunk:
                nxt = issue(ci + 1, 1 - s)
            for cp in pending:
                cp.wait()
            compute(ci, s)
            pending = nxt

    return sc_k(v_pos, u_pos, neg_t, v_emb, u_emb)


def _tc_loss(parts):
    B = parts.shape[0]

    def body(p_ref, o_ref):
        x = p_ref[:]
        pos = jnp.sum(x[:, 0:L], axis=1)
        neg = jnp.sum(x[:, L:2 * L], axis=1)

        def logsig(v):
            return jnp.minimum(v, 0.0) - jnp.log1p(jnp.exp(-jnp.abs(v)))

        total = jnp.sum(logsig(pos)) + jnp.sum(logsig(-neg))
        o_ref[0, 0] = -total / B

    return pl.pallas_call(
        body,
        out_shape=jax.ShapeDtypeStruct((1, 1), jnp.float32),
        out_specs=pl.BlockSpec(memory_space=pltpu.SMEM),
    )(parts)


def kernel(v_pos, u_pos, neg_pos, v_emb, u_emb):
    neg_t = jnp.transpose(neg_pos)
    flat = _sc_gather_dots(v_pos, u_pos, neg_t, v_emb, u_emb)
    parts = flat.reshape(v_pos.shape[0], 2 * L)
    loss2 = _tc_loss(parts)
    return loss2[0, 0]


# speedup vs baseline: 1.0267x; 1.0267x over previous
import functools

import jax
import jax.numpy as jnp
from jax import lax
from jax.experimental import pallas as pl
from jax.experimental.pallas import tpu as pltpu
from jax.experimental.pallas import tpu_sc as plsc

NC = 2
NS = 16
NW = NC * NS
L = 16

DIM = 64
NEG = 20
CHUNK = 128


def _sc_gather_dots(v_pos, u_pos, neg_t, v_emb, u_emb):
    B = v_pos.shape[0]
    bpw = B // NW
    nchunk = bpw // CHUNK
    mesh = plsc.VectorSubcoreMesh(core_axis_name="c", subcore_axis_name="s")

    @functools.partial(
        pl.kernel,
        out_type=jax.ShapeDtypeStruct((B * 2 * L,), jnp.float32),
        mesh=mesh,
        compiler_params=pltpu.CompilerParams(use_tc_tiling_on_sc=False),
        scratch_types=[
            pltpu.VMEM((2, CHUNK), jnp.int32),
            pltpu.VMEM((2, CHUNK), jnp.int32),
            pltpu.VMEM((2, NEG, CHUNK), jnp.int32),
            pltpu.VMEM((2, CHUNK, DIM), jnp.float32),
            pltpu.VMEM((2, CHUNK, DIM), jnp.float32),
            pltpu.VMEM((2, CHUNK, DIM), jnp.float32),
            pltpu.VMEM((CHUNK * 2 * L,), jnp.float32),
            pltpu.SemaphoreType.DMA,
            pltpu.SemaphoreType.DMA,
        ],
    )
    def sc_k(vpos_hbm, upos_hbm, negt_hbm, vemb_hbm, uemb_hbm, out_hbm,
             vidx, uidx, nidx, vbuf, ubuf, acc, po, sem0, sem1):
        wid = lax.axis_index("s") * NC + lax.axis_index("c")
        base_w = wid * bpw
        sems = (sem0, sem1)
        zero = jnp.zeros((L,), jnp.float32)

        def zero_body(c, _):
            for s in range(2):
                for j in range(DIM // L):
                    acc.at[s][c, pl.ds(j * L, L)] = zero
            return 0
        lax.fori_loop(0, CHUNK, zero_body, 0)

        def issue(ci, s):
            base = base_w + ci * CHUNK
            pltpu.sync_copy(vpos_hbm.at[pl.ds(base, CHUNK)], vidx.at[s])
            pltpu.sync_copy(upos_hbm.at[pl.ds(base, CHUNK)], uidx.at[s])
            pltpu.sync_copy(negt_hbm.at[:, pl.ds(base, CHUNK)], nidx.at[s])
            cps = [
                pltpu.async_copy(vemb_hbm.at[vidx.at[s]], vbuf.at[s], sems[s]),
                pltpu.async_copy(uemb_hbm.at[uidx.at[s]], ubuf.at[s], sems[s]),
            ]
            for k in range(NEG):
                cps.append(pltpu.async_copy(
                    uemb_hbm.at[nidx.at[s].at[k]], acc.at[s], sems[s],
                    add=True))
            return cps

        def compute(ci, s):
            base = base_w + ci * CHUNK
            vb, ub, ab = vbuf.at[s], ubuf.at[s], acc.at[s]

            def body(c, _):
                pd = jnp.zeros((L,), jnp.float32)
                nd = jnp.zeros((L,), jnp.float32)
                for j in range(DIM // L):
                    sl = pl.ds(j * L, L)
                    vv = vb[c, sl]
                    pd = pd + vv * ub[c, sl]
                    nd = nd + vv * ab[c, sl]
                    ab[c, sl] = zero
                po.at[pl.ds(c * 2 * L, L)][...] = pd
                po.at[pl.ds(c * 2 * L + L, L)][...] = nd
                return 0
            lax.fori_loop(0, CHUNK, body, 0)
            pltpu.sync_copy(po, out_hbm.at[pl.ds(base * 2 * L,
                                                 CHUNK * 2 * L)])

        pending = issue(0, 0)
        for ci in range(nchunk):
            s = ci % 2
            nxt = None
            if ci + 1 < nchunk:
                nxt = issue(ci + 1, 1 - s)
            for cp in pending:
                cp.wait()
            compute(ci, s)
            pending = nxt

    return sc_k(v_pos, u_pos, neg_t, v_emb, u_emb)


def _tc_loss(parts):
    B = parts.shape[0]

    def body(p_ref, o_ref):
        x = p_ref[:]
        pos = jnp.sum(x[:, 0:L], axis=1)
        neg = jnp.sum(x[:, L:2 * L], axis=1)

        def logsig(v):
            return jnp.minimum(v, 0.0) - jnp.log1p(jnp.exp(-jnp.abs(v)))

        total = jnp.sum(logsig(pos)) + jnp.sum(logsig(-neg))
        o_ref[0, 0] = -total / B

    return pl.pallas_call(
        body,
        out_shape=jax.ShapeDtypeStruct((1, 1), jnp.float32),
        out_specs=pl.BlockSpec(memory_space=pltpu.SMEM),
    )(parts)


def kernel(v_pos, u_pos, neg_pos, v_emb, u_emb):
    neg_t = jnp.transpose(neg_pos)
    flat = _sc_gather_dots(v_pos, u_pos, neg_t, v_emb, u_emb)
    parts = flat.reshape(v_pos.shape[0], 2 * L)
    loss2 = _tc_loss(parts)
    return loss2[0, 0]
